# baseline (speedup 1.0000x reference)
import functools

import jax
import jax.numpy as jnp
from jax import lax
from jax.experimental import pallas as pl
from jax.experimental.pallas import tpu as pltpu
from jax.experimental.pallas import tpu_sc as plsc

_B = 4096
_D = 64
_L = 16


@jax.jit
def _gmf(user_ids, item_ids, user_table, item_table):
    info = plsc.get_sparse_core_info()
    nc, ns = info.num_cores, info.num_subcores
    nw = nc * ns
    b_per_w = _B // nw
    o_per_w = b_per_w // 2

    ut2 = user_table.reshape(-1, 2 * _D)
    it2 = item_table.reshape(-1, 2 * _D)

    mesh = plsc.VectorSubcoreMesh(core_axis_name="c", subcore_axis_name="s")

    @functools.partial(
        pl.kernel,
        mesh=mesh,
        out_type=jax.ShapeDtypeStruct((_B // 2, 2 * _D), jnp.float32),
        scratch_types=[
            pltpu.VMEM((b_per_w,), jnp.int32),
            pltpu.VMEM((b_per_w,), jnp.int32),
            pltpu.VMEM((b_per_w,), jnp.int32),
            pltpu.VMEM((b_per_w,), jnp.int32),
            pltpu.VMEM((b_per_w,), jnp.int32),
            pltpu.VMEM((b_per_w,), jnp.int32),
            pltpu.VMEM((b_per_w, 2 * _D), jnp.float32),
            pltpu.VMEM((b_per_w, 2 * _D), jnp.float32),
            pltpu.VMEM((o_per_w, 2 * _D), jnp.float32),
            pltpu.SemaphoreType.DMA,
            pltpu.SemaphoreType.DMA,
        ],
        compiler_params=pltpu.CompilerParams(needs_layout_passes=False),
    )
    def k(uid_hbm, iid_hbm, utab_hbm, itab_hbm, out_hbm,
          uidx_v, iidx_v, uhidx_v, ihidx_v, ucol_v, icol_v,
          uwide_v, iwide_v, prod_v, semu, semi):
        wid = lax.axis_index("s") * nc + lax.axis_index("c")
        base = wid * b_per_w
        pltpu.sync_copy(uid_hbm.at[pl.ds(base, b_per_w)], uidx_v)
        pltpu.sync_copy(iid_hbm.at[pl.ds(base, b_per_w)], iidx_v)

        def split(cidx, carry):
            s = pl.ds(cidx * _L, _L)
            uvec = uidx_v[s]
            ivec = iidx_v[s]
            uhidx_v[s] = uvec >> 1
            ihidx_v[s] = ivec >> 1
            ucol_v[s] = (uvec & 1) * _D
            icol_v[s] = (ivec & 1) * _D
            return carry

        lax.fori_loop(0, b_per_w // _L, split, 0)

        cu = pltpu.async_copy(utab_hbm.at[uhidx_v], uwide_v, semu)
        ci = pltpu.async_copy(itab_hbm.at[ihidx_v], iwide_v, semi)
        cu.wait()
        ci.wait()

        lanes = lax.iota(jnp.int32, _L)

        def body(cidx, carry):
            cbase = cidx * _L
            ucvec = ucol_v[pl.ds(cbase, _L)]
            icvec = icol_v[pl.ds(cbase, _L)]
            for l in range(_L):
                uc = jnp.sum(jnp.where(lanes == l, ucvec, 0))
                ic = jnp.sum(jnp.where(lanes == l, icvec, 0))
                r = cbase + l
                j = r // 2
                half = (r % 2) * _D
                for c in range(_D // _L):
                    prod_v[j, pl.ds(half + c * _L, _L)] = (
                        uwide_v[r, pl.ds(uc + c * _L, _L)]
                        * iwide_v[r, pl.ds(ic + c * _L, _L)])
            return carry

        lax.fori_loop(0, b_per_w // _L, body, 0)
        pltpu.sync_copy(prod_v, out_hbm.at[pl.ds(wid * o_per_w, o_per_w)])

    out2 = k(user_ids, item_ids, ut2, it2)
    return out2.reshape(_B, _D)


def kernel(user_ids, item_ids, user_table, item_table):
    return _gmf(user_ids, item_ids, user_table, item_table)


# speedup vs baseline: 2.9386x; 2.9386x over previous
import functools

import jax
import jax.numpy as jnp
from jax import lax
from jax.experimental import pallas as pl
from jax.experimental.pallas import tpu as pltpu
from jax.experimental.pallas import tpu_sc as plsc

_B = 4096
_V = 100000
_D = 64
_L = 16


@jax.jit
def _gmf(user_ids, item_ids, user_table, item_table):
    info = plsc.get_sparse_core_info()
    nc, ns = info.num_cores, info.num_subcores
    nw = nc * ns
    d_per_w = _D // nw

    utT = user_table.T
    itT = item_table.T

    mesh = plsc.VectorSubcoreMesh(core_axis_name="c", subcore_axis_name="s")

    @functools.partial(
        pl.kernel,
        mesh=mesh,
        out_type=jax.ShapeDtypeStruct((_D, _B), jnp.float32),
        scratch_types=[
            pltpu.VMEM((_B,), jnp.int32),
            pltpu.VMEM((_B,), jnp.int32),
            pltpu.VMEM((1, _V), jnp.float32),
            pltpu.VMEM((_B,), jnp.float32),
            pltpu.VMEM((1, _B), jnp.float32),
        ],
        compiler_params=pltpu.CompilerParams(needs_layout_passes=False),
    )
    def k(uid_hbm, iid_hbm, utT_hbm, itT_hbm, outT_hbm,
          uids_v, iids_v, col_v, ugath_v, orow_v):
        wid = lax.axis_index("s") * nc + lax.axis_index("c")
        pltpu.sync_copy(uid_hbm, uids_v)
        pltpu.sync_copy(iid_hbm, iids_v)

        zeros = jnp.zeros((_L,), jnp.int32)

        for dd in range(d_per_w):
            d = wid * d_per_w + dd
            pltpu.sync_copy(utT_hbm.at[pl.ds(d, 1), :], col_v)

            def gath_u(kk, carry):
                s = pl.ds(kk * _L, _L)
                ugath_v[s] = plsc.load_gather(col_v, [zeros, uids_v[s]])
                return carry

            lax.fori_loop(0, _B // _L, gath_u, 0)
            pltpu.sync_copy(itT_hbm.at[pl.ds(d, 1), :], col_v)

            def gath_i(kk, carry):
                s = pl.ds(kk * _L, _L)
                orow_v[0, s] = ugath_v[s] * plsc.load_gather(
                    col_v, [zeros, iids_v[s]])
                return carry

            lax.fori_loop(0, _B // _L, gath_i, 0)
            pltpu.sync_copy(orow_v, outT_hbm.at[pl.ds(d, 1), :])

    outT = k(user_ids, item_ids, utT, itT)
    return outT.T


def kernel(user_ids, item_ids, user_table, item_table):
    return _gmf(user_ids, item_ids, user_table, item_table)
